# baseline (speedup 1.0000x reference)
import jax
import jax.numpy as jnp
from jax import lax
from jax.experimental import pallas as pl
from jax.experimental.pallas import tpu as pltpu
from jax.experimental.pallas import tpu_sc as plsc

_NW = 32
_K = 16

_DNUMS = lax.GatherDimensionNumbers(
    offset_dims=(), collapsed_slice_dims=(0,), start_index_map=(0,))


def _g16(v, idx):
    return lax.gather(v, idx[:, None], dimension_numbers=_DNUMS,
                      slice_sizes=(1,),
                      mode=lax.GatherScatterMode.PROMISE_IN_BOUNDS)


def _prefix16(x, iota):
    for sh in (1, 2, 4, 8):
        g = _g16(x, jnp.maximum(iota - sh, 0))
        x = x + jnp.where(iota >= sh, g, 0)
    return x


def _rank_select(cs, tgt):
    src = jnp.zeros((16,), jnp.int32)
    for step in (8, 4, 2, 1):
        c = _g16(cs, src + (step - 1))
        src = src + jnp.where(c <= tgt, step, 0)
    return src


def _sc_body(mask_hbm, t_hbm, out_hbm, mask_v, kept_v, miss_v,
             buf_a, buf_b, buf_c, sem_g, sem_s, sem_z):
    rows, d = t_hbm.shape
    rpw = rows // _NW
    wid = lax.axis_index("s") * 2 + lax.axis_index("c")
    base = wid * rpw

    pltpu.sync_copy(mask_hbm.at[pl.ds(wid * (rpw // 4), rpw // 4)], mask_v)

    iota = lax.iota(jnp.int32, 16)

    def compact(g, carry):
        nk, nm = carry
        w = mask_v[pl.ds(g * 16, 16)]
        for j in range(4):
            mi = (w >> (8 * j)) & 1
            idxv = base + g * 64 + iota * 4 + j
            csk = _prefix16(mi, iota)
            ck = csk[15]
            csm = iota + 1 - csk
            kept_v[pl.ds(nk, 16)] = _g16(idxv, _rank_select(csk, iota))
            miss_v[pl.ds(nm, 16)] = _g16(idxv, _rank_select(csm, iota))
            nk = nk + ck
            nm = nm + (16 - ck)
        return (nk, nm)

    nk, nm = lax.fori_loop(0, rpw // 64, compact,
                           (jnp.int32(0), jnp.int32(0)))

    k0 = jnp.where(nk > 0, kept_v[pl.ds(0, 16)][0], base)
    m0 = jnp.where(nm > 0, miss_v[pl.ds(0, 16)][0], base)
    kept_v[pl.ds(nk, 16)] = jnp.zeros((16,), jnp.int32) + k0
    miss_v[pl.ds(nm, 16)] = jnp.zeros((16,), jnp.int32) + m0

    nck = (nk + _K - 1) // _K
    ncz = (nm + _K - 1) // _K

    def _kidx(c):
        return kept_v[pl.ds(c * _K, _K)]

    def _gather_to(c, buf):
        pltpu.async_copy(t_hbm.at[_kidx(c)], buf, sem_g)

    def _scatter_from(c, buf):
        pltpu.async_copy(buf, out_hbm.at[_kidx(c)], sem_s)

    def _ring(c, fn):
        r = c % 3

        @pl.when(r == 0)
        def _r0():
            fn(buf_a)

        @pl.when(r == 1)
        def _r1():
            fn(buf_b)

        @pl.when(r == 2)
        def _r2():
            fn(buf_c)

    @pl.when(nck > 0)
    def _p0():
        _gather_to(0, buf_a)

    @pl.when(nck > 1)
    def _p1():
        _gather_to(1, buf_b)

    def body(c, carry):
        pltpu.make_async_copy(t_hbm.at[pl.ds(0, _K)], buf_a, sem_g).wait()
        _ring(c, lambda buf: _scatter_from(c, buf))

        @pl.when(c + 2 < nck)
        def _issue():
            @pl.when(c >= 1)
            def _w():
                pltpu.make_async_copy(
                    buf_a, out_hbm.at[pl.ds(0, _K)], sem_s).wait()
            _ring(c + 2, lambda buf: _gather_to(c + 2, buf))

        return carry

    lax.fori_loop(0, nck, body, 0)

    def sdrain(c, carry):
        pltpu.make_async_copy(buf_a, out_hbm.at[pl.ds(0, _K)], sem_s).wait()
        return carry

    lax.fori_loop(0, jnp.minimum(nck, 3), sdrain, 0)

    zv = jnp.zeros((16,), jnp.float32)

    def zinit_row(i, carry):
        def zinit_col(j, carry2):
            buf_c[i, pl.ds(j * 16, 16)] = zv
            return carry2
        return lax.fori_loop(0, d // 16, zinit_col, carry)

    lax.fori_loop(0, _K, zinit_row, 0)

    def zbody(c, carry):
        pltpu.async_copy(
            buf_c, out_hbm.at[miss_v[pl.ds(c * _K, _K)]], sem_z)
        return carry

    lax.fori_loop(0, ncz, zbody, 0)

    def zdrain(c, carry):
        pltpu.make_async_copy(buf_c, out_hbm.at[pl.ds(0, _K)], sem_z).wait()
        return carry

    lax.fori_loop(0, ncz, zdrain, 0)


def kernel(tensor, mask):
    B, S, D = tensor.shape
    rows = B * S
    t2d = tensor.reshape(rows, D)
    m1w = lax.bitcast_convert_type(
        mask.reshape(rows // 4, 4).view(jnp.uint8), jnp.int32)
    rpw = rows // _NW

    kfn = pl.kernel(
        _sc_body,
        out_type=jax.ShapeDtypeStruct((rows, D), jnp.float32),
        mesh=plsc.VectorSubcoreMesh(core_axis_name="c", subcore_axis_name="s"),
        scratch_types=[
            pltpu.VMEM((rpw // 4,), jnp.int32),
            pltpu.VMEM((rpw + 16,), jnp.int32),
            pltpu.VMEM((rpw + 16,), jnp.int32),
            pltpu.VMEM((_K, D), jnp.float32),
            pltpu.VMEM((_K, D), jnp.float32),
            pltpu.VMEM((_K, D), jnp.float32),
            pltpu.SemaphoreType.DMA,
            pltpu.SemaphoreType.DMA,
            pltpu.SemaphoreType.DMA,
        ],
    )
    return kfn(m1w, t2d).reshape(B, S, D)


# speedup vs baseline: 1.0303x; 1.0303x over previous
import jax
import jax.numpy as jnp
from jax import lax
from jax.experimental import pallas as pl
from jax.experimental.pallas import tpu as pltpu
from jax.experimental.pallas import tpu_sc as plsc

_NW = 32
_K = 16

_DNUMS = lax.GatherDimensionNumbers(
    offset_dims=(), collapsed_slice_dims=(0,), start_index_map=(0,))


def _g16(v, idx):
    return lax.gather(v, idx[:, None], dimension_numbers=_DNUMS,
                      slice_sizes=(1,),
                      mode=lax.GatherScatterMode.PROMISE_IN_BOUNDS)


def _prefix16(x, iota):
    for sh in (1, 2, 4, 8):
        g = _g16(x, jnp.maximum(iota - sh, 0))
        x = x + jnp.where(iota >= sh, g, 0)
    return x


def _rank_select(cs, tgt):
    src = jnp.zeros((16,), jnp.int32)
    for step in (8, 4, 2, 1):
        c = _g16(cs, src + (step - 1))
        src = src + jnp.where(c <= tgt, step, 0)
    return src


def _sc_body(mask_hbm, t_hbm, out_hbm, mask_v, kept_v, miss_v,
             buf_a, buf_b, buf_c, sem_g, sem_s, sem_z):
    rows, d = t_hbm.shape
    rpw = rows // _NW
    wid = lax.axis_index("s") * 2 + lax.axis_index("c")
    base = wid * rpw

    pltpu.sync_copy(mask_hbm.at[pl.ds(base, rpw)], mask_v)

    iota = lax.iota(jnp.int32, 16)

    def _kidx(c):
        return kept_v[pl.ds(c * _K, _K)]

    def _gather_to(c, buf):
        pltpu.async_copy(t_hbm.at[_kidx(c)], buf, sem_g)

    def _scatter_from(c, buf):
        pltpu.async_copy(buf, out_hbm.at[_kidx(c)], sem_s)

    def compact(g, carry):
        nk, nm, primed = carry
        m16 = mask_v[pl.ds(g * 16, 16)]
        idxv = base + g * 16 + iota
        mi = jnp.where(m16 != 0, 1, 0)
        csk = _prefix16(mi, iota)
        ck = csk[15]
        csm = iota + 1 - csk
        kept_v[pl.ds(nk, 16)] = _g16(idxv, _rank_select(csk, iota))
        miss_v[pl.ds(nm, 16)] = _g16(idxv, _rank_select(csm, iota))
        nk = nk + ck
        do_prime = (primed == 0) & (nk >= 48)

        @pl.when(do_prime)
        def _p():
            _gather_to(0, buf_a)
            _gather_to(1, buf_b)

        return (nk, nm + (16 - ck), jnp.where(do_prime, 1, primed))

    nk, nm, primed = lax.fori_loop(0, rpw // 16, compact,
                                   (jnp.int32(0), jnp.int32(0),
                                    jnp.int32(0)))

    k0 = jnp.where(nk > 0, kept_v[pl.ds(0, 16)][0], base)
    m0 = jnp.where(nm > 0, miss_v[pl.ds(0, 16)][0], base)
    kept_v[pl.ds(nk, 16)] = jnp.zeros((16,), jnp.int32) + k0
    miss_v[pl.ds(nm, 16)] = jnp.zeros((16,), jnp.int32) + m0

    nck = (nk + _K - 1) // _K
    ncz = (nm + _K - 1) // _K

    def _ring(c, fn):
        r = c % 3

        @pl.when(r == 0)
        def _r0():
            fn(buf_a)

        @pl.when(r == 1)
        def _r1():
            fn(buf_b)

        @pl.when(r == 2)
        def _r2():
            fn(buf_c)

    @pl.when((primed == 0) & (nck > 0))
    def _p0():
        _gather_to(0, buf_a)

    @pl.when((primed == 0) & (nck > 1))
    def _p1():
        _gather_to(1, buf_b)

    def body(c, carry):
        pltpu.make_async_copy(t_hbm.at[pl.ds(0, _K)], buf_a, sem_g).wait()
        _ring(c, lambda buf: _scatter_from(c, buf))

        @pl.when(c + 2 < nck)
        def _issue():
            @pl.when(c >= 1)
            def _w():
                pltpu.make_async_copy(
                    buf_a, out_hbm.at[pl.ds(0, _K)], sem_s).wait()
            _ring(c + 2, lambda buf: _gather_to(c + 2, buf))

        return carry

    lax.fori_loop(0, nck, body, 0)

    def sdrain(c, carry):
        pltpu.make_async_copy(buf_a, out_hbm.at[pl.ds(0, _K)], sem_s).wait()
        return carry

    lax.fori_loop(0, jnp.minimum(nck, 3), sdrain, 0)

    zv = jnp.zeros((16,), jnp.float32)

    def zinit_row(i, carry):
        def zinit_col(j, carry2):
            buf_c[i, pl.ds(j * 16, 16)] = zv
            return carry2
        return lax.fori_loop(0, d // 16, zinit_col, carry)

    lax.fori_loop(0, _K, zinit_row, 0)

    def zbody(c, carry):
        pltpu.async_copy(
            buf_c, out_hbm.at[miss_v[pl.ds(c * _K, _K)]], sem_z)
        return carry

    lax.fori_loop(0, ncz, zbody, 0)

    def zdrain(c, carry):
        pltpu.make_async_copy(buf_c, out_hbm.at[pl.ds(0, _K)], sem_z).wait()
        return carry

    lax.fori_loop(0, ncz, zdrain, 0)


def kernel(tensor, mask):
    B, S, D = tensor.shape
    rows = B * S
    t2d = tensor.reshape(rows, D)
    m1d = mask.astype(jnp.int32).reshape(rows)
    rpw = rows // _NW

    kfn = pl.kernel(
        _sc_body,
        out_type=jax.ShapeDtypeStruct((rows, D), jnp.float32),
        mesh=plsc.VectorSubcoreMesh(core_axis_name="c", subcore_axis_name="s"),
        scratch_types=[
            pltpu.VMEM((rpw,), jnp.int32),
            pltpu.VMEM((rpw + 16,), jnp.int32),
            pltpu.VMEM((rpw + 16,), jnp.int32),
            pltpu.VMEM((_K, D), jnp.float32),
            pltpu.VMEM((_K, D), jnp.float32),
            pltpu.VMEM((_K, D), jnp.float32),
            pltpu.SemaphoreType.DMA,
            pltpu.SemaphoreType.DMA,
            pltpu.SemaphoreType.DMA,
        ],
    )
    return kfn(m1d, t2d).reshape(B, S, D)
